# Initial kernel scaffold; baseline (speedup 1.0000x reference)
import jax
import jax.numpy as jnp
from jax.experimental import pallas as pl


def kernel(x, up_index, up_orient, down_index, down_orient, batch, W_up_0, W_down_0, W_0, W_up_1, W_down_1, W_1, W_up_2, W_down_2, W_2, lin1_W, lin1_b, lin2_W, lin2_b):
    raise NotImplementedError("write your pallas kernel here")


# speedup vs baseline: 4.2072x; 4.2072x over previous
import functools

import jax
import jax.numpy as jnp
from jax import lax
from jax.experimental import pallas as pl
from jax.experimental.pallas import tpu as pltpu
from jax.experimental.pallas import tpu_sc as plsc

NC = 2
NS = 16
K = 128


def _sc_scatter_call(table, gidx, didx, zeros, tpw):
    n, h = zeros.shape
    rpt = (n // NS) & ~7
    tail = n - NS * rpt
    mesh = plsc.VectorSubcoreMesh(core_axis_name="c", subcore_axis_name="s")

    @functools.partial(
        pl.kernel,
        mesh=mesh,
        out_type=jax.ShapeDtypeStruct((NC, n, h), jnp.float32),
        scratch_types=[
            pltpu.VMEM_SHARED((n, h), jnp.float32),
            pltpu.VMEM((K,), jnp.int32),
            pltpu.VMEM((K,), jnp.int32),
            pltpu.VMEM((K, h), jnp.float32),
        ],
    )
    def sc_kernel(table_hbm, gidx_hbm, didx_hbm, zeros_hbm, out_hbm,
                  accum, gbuf, dbuf, rows):
        cid = lax.axis_index("c")
        sid = lax.axis_index("s")
        r0 = sid * rpt
        pltpu.sync_copy(zeros_hbm.at[pl.ds(r0, rpt)], accum.at[pl.ds(r0, rpt)])
        if tail:
            @pl.when(sid == NS - 1)
            def _():
                pltpu.sync_copy(zeros_hbm.at[pl.ds(NS * rpt, tail)],
                                accum.at[pl.ds(NS * rpt, tail)])
        plsc.subcore_barrier()
        tile_base = (cid * NS + sid) * (tpw * K)

        @pl.loop(0, tpw)
        def _(j):
            base = tile_base + j * K
            pltpu.sync_copy(gidx_hbm.at[pl.ds(base, K)], gbuf)
            pltpu.sync_copy(didx_hbm.at[pl.ds(base, K)], dbuf)
            pltpu.sync_copy(table_hbm.at[gbuf], rows)
            pltpu.sync_copy(rows, accum.at[dbuf], add=True)

        plsc.subcore_barrier()
        pltpu.sync_copy(accum.at[pl.ds(r0, rpt)],
                        out_hbm.at[cid, pl.ds(r0, rpt)])
        if tail:
            @pl.when(sid == NS - 1)
            def _():
                pltpu.sync_copy(accum.at[pl.ds(NS * rpt, tail)],
                                out_hbm.at[cid, pl.ds(NS * rpt, tail)])

    return sc_kernel(table, gidx, didx, zeros)


def _tc_layer_call(xb, agg, wu, wd, w, blk):
    n, d = xb.shape
    h = wu.shape[1]
    nb = n // blk
    has_agg = agg is not None

    def body(*refs):
        if has_agg:
            xr, ar, wur, wdr, wr, t_ref, base_ref = refs
            x = xr[...] + ar[0] + ar[1]
        else:
            xr, wur, wdr, wr, t_ref, base_ref = refs
            x = xr[...]
        u = jnp.dot(x, wur[...], preferred_element_type=jnp.float32)
        v = jnp.dot(x, wdr[...], preferred_element_type=jnp.float32)
        t_ref[0] = u
        t_ref[1] = v
        t_ref[2] = -u
        t_ref[3] = -v
        t_ref[4] = jnp.zeros((blk, h), jnp.float32)
        base_ref[...] = jnp.dot(x, wr[...], preferred_element_type=jnp.float32)

    in_specs = [pl.BlockSpec((blk, d), lambda i: (i, 0))]
    args = [xb]
    if has_agg:
        in_specs.append(pl.BlockSpec((NC, blk, h), lambda i: (0, i, 0)))
        args.append(agg)
    in_specs += [pl.BlockSpec((d, h), lambda i: (0, 0))] * 3
    args += [wu, wd, w]

    return pl.pallas_call(
        body,
        grid=(nb,),
        in_specs=in_specs,
        out_specs=[
            pl.BlockSpec((5, blk, h), lambda i: (0, i, 0)),
            pl.BlockSpec((blk, h), lambda i: (i, 0)),
        ],
        out_shape=[
            jax.ShapeDtypeStruct((5, n, h), jnp.float32),
            jax.ShapeDtypeStruct((n, h), jnp.float32),
        ],
    )(*args)


def _tc_readout_call(base, agg, batch2d, w1, b1, w2, b2, nseg, blk):
    n, h = base.shape
    c = w2.shape[1]
    nb = n // blk

    def body(base_ref, a_ref, bt_ref, w1_ref, b1_ref, w2_ref, b2_ref,
             out_ref, pooled_ref):
        i = pl.program_id(0)
        x = base_ref[...] + a_ref[0] + a_ref[1]
        xa = jnp.abs(x)
        seg = bt_ref[...]
        onehot = (seg == lax.broadcasted_iota(jnp.int32, (blk, nseg), 1)
                  ).astype(jnp.float32)
        part = lax.dot_general(onehot, xa, (((0,), (0,)), ((), ())),
                               preferred_element_type=jnp.float32)

        @pl.when(i == 0)
        def _():
            pooled_ref[...] = part

        @pl.when(i > 0)
        def _():
            pooled_ref[...] += part

        @pl.when(i == nb - 1)
        def _():
            hdn = jnp.maximum(
                jnp.dot(pooled_ref[...], w1_ref[...],
                        preferred_element_type=jnp.float32) + b1_ref[...], 0.0)
            out_ref[...] = jnp.dot(hdn, w2_ref[...],
                                   preferred_element_type=jnp.float32) + b2_ref[...]

    return pl.pallas_call(
        body,
        grid=(nb,),
        in_specs=[
            pl.BlockSpec((blk, h), lambda i: (i, 0)),
            pl.BlockSpec((NC, blk, h), lambda i: (0, i, 0)),
            pl.BlockSpec((blk, 1), lambda i: (i, 0)),
            pl.BlockSpec((h, h), lambda i: (0, 0)),
            pl.BlockSpec((1, h), lambda i: (0, 0)),
            pl.BlockSpec((h, c), lambda i: (0, 0)),
            pl.BlockSpec((1, c), lambda i: (0, 0)),
        ],
        out_specs=pl.BlockSpec((nseg, c), lambda i: (0, 0)),
        out_shape=jax.ShapeDtypeStruct((nseg, c), jnp.float32),
        scratch_shapes=[pltpu.VMEM((nseg, h), jnp.float32)],
    )(base, agg, batch2d, w1, b1, w2, b2)


def kernel(x, up_index, up_orient, down_index, down_orient, batch,
           W_up_0, W_down_0, W_0, W_up_1, W_down_1, W_1,
           W_up_2, W_down_2, W_2, lin1_W, lin1_b, lin2_W, lin2_b):
    n, d = x.shape
    h = W_0.shape[1]
    e = up_index.shape[1]
    nseg = 64
    c = lin2_W.shape[1]
    blk = 1000

    up_g = jnp.where(up_orient > 0, up_index[0],
                     jnp.where(up_orient < 0, up_index[0] + 2 * n,
                               up_index[0] + 4 * n))
    dn_g = jnp.where(down_orient > 0, down_index[0] + n,
                     jnp.where(down_orient < 0, down_index[0] + 3 * n,
                               down_index[0] + 4 * n))
    gidx = jnp.concatenate([up_g, dn_g]).astype(jnp.int32)
    didx = jnp.concatenate([up_index[1], down_index[1]]).astype(jnp.int32)

    ep = 2 * e
    tiles = NC * NS
    tpw = -(-ep // (tiles * K))
    epad = tiles * tpw * K
    pad = epad - ep
    if pad:
        gidx = jnp.concatenate([gidx, jnp.full((pad,), 4 * n, jnp.int32)])
        didx = jnp.concatenate([didx, jnp.zeros((pad,), jnp.int32)])

    zeros = jnp.zeros((n, h), jnp.float32)
    batch2d = batch.astype(jnp.int32).reshape(n, 1)

    wus = [W_up_0, W_up_1, W_up_2]
    wds = [W_down_0, W_down_1, W_down_2]
    ws = [W_0, W_1, W_2]

    xb, agg = x, None
    for l in range(3):
        t5, base = _tc_layer_call(xb, agg, wus[l], wds[l], ws[l], blk)
        agg = _sc_scatter_call(t5.reshape(5 * n, h), gidx, didx, zeros, tpw)
        xb = base

    return _tc_readout_call(xb, agg, batch2d, lin1_W,
                            lin1_b.reshape(1, h), lin2_W,
                            lin2_b.reshape(1, c), nseg, blk)


# speedup vs baseline: 4.2894x; 1.0195x over previous
import functools

import jax
import jax.numpy as jnp
from jax import lax
from jax.experimental import pallas as pl
from jax.experimental.pallas import tpu as pltpu
from jax.experimental.pallas import tpu_sc as plsc

NC = 2
NS = 16
K = 128


def _sc_scatter_call(table, idxpair, zeros, tpw):
    n, h = zeros.shape
    rpt = (n // NS) & ~7
    tail = n - NS * rpt
    npairs = tpw // 2
    mesh = plsc.VectorSubcoreMesh(core_axis_name="c", subcore_axis_name="s")

    @functools.partial(
        pl.kernel,
        mesh=mesh,
        out_type=jax.ShapeDtypeStruct((NC, n, h), jnp.float32),
        scratch_types=[
            pltpu.VMEM_SHARED((n, h), jnp.float32),
            pltpu.VMEM((2, K), jnp.int32),
            pltpu.VMEM((2, K), jnp.int32),
            pltpu.VMEM((K, h), jnp.float32),
            pltpu.VMEM((K, h), jnp.float32),
            pltpu.SemaphoreType.DMA,
            pltpu.SemaphoreType.DMA,
        ],
    )
    def sc_kernel(table_hbm, idx_hbm, zeros_hbm, out_hbm,
                  accum, ibuf0, ibuf1, rows0, rows1, sem0, sem1):
        cid = lax.axis_index("c")
        sid = lax.axis_index("s")
        r0 = sid * rpt
        pltpu.sync_copy(zeros_hbm.at[pl.ds(r0, rpt)], accum.at[pl.ds(r0, rpt)])
        if tail:
            @pl.when(sid == NS - 1)
            def _():
                pltpu.sync_copy(zeros_hbm.at[pl.ds(NS * rpt, tail)],
                                accum.at[pl.ds(NS * rpt, tail)])
        plsc.subcore_barrier()
        c0 = (cid * NS + sid) * tpw

        pltpu.sync_copy(idx_hbm.at[c0], ibuf0)
        pltpu.async_copy(table_hbm.at[ibuf0.at[0]], rows0, sem0)

        @pl.loop(0, npairs)
        def _(j):
            c = c0 + 2 * j
            pltpu.sync_copy(idx_hbm.at[c + 1], ibuf1)
            pltpu.async_copy(table_hbm.at[ibuf1.at[0]], rows1, sem1)
            pltpu.make_async_copy(table_hbm.at[ibuf0.at[0]], rows0, sem0).wait()
            pltpu.sync_copy(rows0, accum.at[ibuf0.at[1]], add=True)

            @pl.when(j < npairs - 1)
            def _():
                pltpu.sync_copy(idx_hbm.at[c + 2], ibuf0)
                pltpu.async_copy(table_hbm.at[ibuf0.at[0]], rows0, sem0)

            pltpu.make_async_copy(table_hbm.at[ibuf1.at[0]], rows1, sem1).wait()
            pltpu.sync_copy(rows1, accum.at[ibuf1.at[1]], add=True)

        plsc.subcore_barrier()
        pltpu.sync_copy(accum.at[pl.ds(r0, rpt)],
                        out_hbm.at[cid, pl.ds(r0, rpt)])
        if tail:
            @pl.when(sid == NS - 1)
            def _():
                pltpu.sync_copy(accum.at[pl.ds(NS * rpt, tail)],
                                out_hbm.at[cid, pl.ds(NS * rpt, tail)])

    return sc_kernel(table, idxpair, zeros)


def _tc_layer_call(xb, agg, wu, wd, w, blk):
    n, d = xb.shape
    h = wu.shape[1]
    nb = n // blk
    has_agg = agg is not None

    def body(*refs):
        if has_agg:
            xr, ar, wur, wdr, wr, t_ref, base_ref = refs
            x = xr[...] + ar[0] + ar[1]
        else:
            xr, wur, wdr, wr, t_ref, base_ref = refs
            x = xr[...]
        u = jnp.dot(x, wur[...], preferred_element_type=jnp.float32)
        v = jnp.dot(x, wdr[...], preferred_element_type=jnp.float32)
        t_ref[0] = u
        t_ref[1] = v
        t_ref[2] = -u
        t_ref[3] = -v
        t_ref[4] = jnp.zeros((blk, h), jnp.float32)
        base_ref[...] = jnp.dot(x, wr[...], preferred_element_type=jnp.float32)

    in_specs = [pl.BlockSpec((blk, d), lambda i: (i, 0))]
    args = [xb]
    if has_agg:
        in_specs.append(pl.BlockSpec((NC, blk, h), lambda i: (0, i, 0)))
        args.append(agg)
    in_specs += [pl.BlockSpec((d, h), lambda i: (0, 0))] * 3
    args += [wu, wd, w]

    return pl.pallas_call(
        body,
        grid=(nb,),
        in_specs=in_specs,
        out_specs=[
            pl.BlockSpec((5, blk, h), lambda i: (0, i, 0)),
            pl.BlockSpec((blk, h), lambda i: (i, 0)),
        ],
        out_shape=[
            jax.ShapeDtypeStruct((5, n, h), jnp.float32),
            jax.ShapeDtypeStruct((n, h), jnp.float32),
        ],
    )(*args)


def _tc_readout_call(base, agg, batch2d, w1, b1, w2, b2, nseg, blk):
    n, h = base.shape
    c = w2.shape[1]
    nb = n // blk

    def body(base_ref, a_ref, bt_ref, w1_ref, b1_ref, w2_ref, b2_ref,
             out_ref, pooled_ref):
        i = pl.program_id(0)
        x = base_ref[...] + a_ref[0] + a_ref[1]
        xa = jnp.abs(x)
        seg = bt_ref[...]
        onehot = (seg == lax.broadcasted_iota(jnp.int32, (blk, nseg), 1)
                  ).astype(jnp.float32)
        part = lax.dot_general(onehot, xa, (((0,), (0,)), ((), ())),
                               preferred_element_type=jnp.float32)

        @pl.when(i == 0)
        def _():
            pooled_ref[...] = part

        @pl.when(i > 0)
        def _():
            pooled_ref[...] += part

        @pl.when(i == nb - 1)
        def _():
            hdn = jnp.maximum(
                jnp.dot(pooled_ref[...], w1_ref[...],
                        preferred_element_type=jnp.float32) + b1_ref[...], 0.0)
            out_ref[...] = jnp.dot(hdn, w2_ref[...],
                                   preferred_element_type=jnp.float32) + b2_ref[...]

    return pl.pallas_call(
        body,
        grid=(nb,),
        in_specs=[
            pl.BlockSpec((blk, h), lambda i: (i, 0)),
            pl.BlockSpec((NC, blk, h), lambda i: (0, i, 0)),
            pl.BlockSpec((blk, 1), lambda i: (i, 0)),
            pl.BlockSpec((h, h), lambda i: (0, 0)),
            pl.BlockSpec((1, h), lambda i: (0, 0)),
            pl.BlockSpec((h, c), lambda i: (0, 0)),
            pl.BlockSpec((1, c), lambda i: (0, 0)),
        ],
        out_specs=pl.BlockSpec((nseg, c), lambda i: (0, 0)),
        out_shape=jax.ShapeDtypeStruct((nseg, c), jnp.float32),
        scratch_shapes=[pltpu.VMEM((nseg, h), jnp.float32)],
    )(base, agg, batch2d, w1, b1, w2, b2)


def kernel(x, up_index, up_orient, down_index, down_orient, batch,
           W_up_0, W_down_0, W_0, W_up_1, W_down_1, W_1,
           W_up_2, W_down_2, W_2, lin1_W, lin1_b, lin2_W, lin2_b):
    n, d = x.shape
    h = W_0.shape[1]
    e = up_index.shape[1]
    nseg = 64
    c = lin2_W.shape[1]
    blk = 1000

    up_g = jnp.where(up_orient > 0, up_index[0],
                     jnp.where(up_orient < 0, up_index[0] + 2 * n,
                               up_index[0] + 4 * n))
    dn_g = jnp.where(down_orient > 0, down_index[0] + n,
                     jnp.where(down_orient < 0, down_index[0] + 3 * n,
                               down_index[0] + 4 * n))
    gidx = jnp.concatenate([up_g, dn_g]).astype(jnp.int32)
    didx = jnp.concatenate([up_index[1], down_index[1]]).astype(jnp.int32)

    ep = 2 * e
    tiles = NC * NS
    tpw = -(-ep // (tiles * K))
    tpw += tpw % 2
    epad = tiles * tpw * K
    pad = epad - ep
    if pad:
        gidx = jnp.concatenate([gidx, jnp.full((pad,), 4 * n, jnp.int32)])
        didx = jnp.concatenate([didx, jnp.zeros((pad,), jnp.int32)])
    idxpair = jnp.stack([gidx.reshape(-1, K), didx.reshape(-1, K)], axis=1)

    zeros = jnp.zeros((n, h), jnp.float32)
    batch2d = batch.astype(jnp.int32).reshape(n, 1)

    wus = [W_up_0, W_up_1, W_up_2]
    wds = [W_down_0, W_down_1, W_down_2]
    ws = [W_0, W_1, W_2]

    xb, agg = x, None
    for l in range(3):
        t5, base = _tc_layer_call(xb, agg, wus[l], wds[l], ws[l], blk)
        agg = _sc_scatter_call(t5.reshape(5 * n, h), idxpair, zeros, tpw)
        xb = base

    return _tc_readout_call(xb, agg, batch2d, lin1_W,
                            lin1_b.reshape(1, h), lin2_W,
                            lin2_b.reshape(1, c), nseg, blk)
